# Initial kernel scaffold; baseline (speedup 1.0000x reference)
import jax
import jax.numpy as jnp
from jax.experimental import pallas as pl


def kernel(x, edge_index, W_lin, b_lin, W_gat, att_src, att_dst, bias_gat, W_y1, b_y1, W_y0, b_y0, Wc1, bc1, Wc2, bc2):
    raise NotImplementedError("write your pallas kernel here")


# speedup vs baseline: 25.1306x; 25.1306x over previous
import jax
import jax.numpy as jnp
from jax import lax
from jax.experimental import pallas as pl
from jax.experimental.pallas import tpu as pltpu
from jax.experimental.pallas import tpu_sc as plsc

N = 50000
E = 800000
F = 128
D = 16
H = 8

BN = 2000
GRID = N // BN
EB = 80
NBLK_TOT = E // EB
NBLK = NBLK_TOT // 16
RCHUNK = 200
NCHUNK = N // RCHUNK
KCH = (NCHUNK + 15) // 16


def _leaky(x):
    return jnp.maximum(x, 0.2 * x)


def _prep_body(x_ref, wlt_ref, bl_ref, wgt_ref, ams_ref, amd_ref,
               g_ref, ad_ref, hp_ref, ws_ref):
    h = jnp.dot(x_ref[...], wlt_ref[...],
                preferred_element_type=jnp.float32) + bl_ref[...]
    hp = jnp.dot(h, wgt_ref[...], preferred_element_type=jnp.float32)
    asrc = jnp.dot(hp, ams_ref[...], preferred_element_type=jnp.float32)
    adst = jnp.dot(hp, amd_ref[...], preferred_element_type=jnp.float32)
    hp_ref[...] = hp
    ws_ref[...] = jnp.exp(_leaky(asrc + adst))
    ones = jnp.ones((1, D), dtype=jnp.float32)
    for hh in range(H):
        g_ref[hh, :, 0:D] = hp[:, hh * D:(hh + 1) * D]
        g_ref[hh, :, D:2 * D] = asrc[:, hh:hh + 1] * ones
        ad_ref[hh, :, :] = adst[:, hh:hh + 1] * ones


def _prep(x, wlt, bl, wgt, ams, amd):
    return pl.pallas_call(
        _prep_body,
        grid=(GRID,),
        in_specs=[
            pl.BlockSpec((BN, F), lambda i: (i, 0)),
            pl.BlockSpec((F, D), lambda i: (0, 0)),
            pl.BlockSpec((1, D), lambda i: (0, 0)),
            pl.BlockSpec((D, H * D), lambda i: (0, 0)),
            pl.BlockSpec((F, H), lambda i: (0, 0)),
            pl.BlockSpec((F, H), lambda i: (0, 0)),
        ],
        out_specs=[
            pl.BlockSpec((H, BN, 2 * D), lambda i: (0, i, 0)),
            pl.BlockSpec((H, BN, D), lambda i: (0, i, 0)),
            pl.BlockSpec((BN, H * D), lambda i: (i, 0)),
            pl.BlockSpec((BN, H), lambda i: (i, 0)),
        ],
        out_shape=[
            jax.ShapeDtypeStruct((H, N, 2 * D), jnp.float32),
            jax.ShapeDtypeStruct((H, N, D), jnp.float32),
            jax.ShapeDtypeStruct((N, H * D), jnp.float32),
            jax.ShapeDtypeStruct((N, H), jnp.float32),
        ],
    )(x, wlt, bl, wgt, ams, amd)


def _edge_kernel_body(g_hbm, ad_hbm, src_hbm, dst_hbm, u_hbm,
                      sidx, didx, dadj, gbuf, abuf, sbuf, stage, zbuf,
                      slab, sem1, sem2):
    cid = lax.axis_index("c")
    sid = lax.axis_index("s")

    for j in range(RCHUNK):
        zbuf[j, 0:D] = jnp.zeros((D,), jnp.float32)
        zbuf[j, D:2 * D] = jnp.zeros((D,), jnp.float32)
    for k in range(KCH):
        chunk = sid + 16 * k

        @pl.when(chunk < NCHUNK)
        def _():
            pltpu.sync_copy(zbuf, slab.at[pl.ds(chunk * RCHUNK, RCHUNK)])
    plsc.subcore_barrier()

    def round_body(r, carry):
        head = cid * 4 + r
        hoff = head * N

        def edge_blk(b, c):
            ebase = (b * 16 + sid) * EB
            pltpu.sync_copy(src_hbm.at[pl.ds(ebase, EB)], sidx)
            pltpu.sync_copy(dst_hbm.at[pl.ds(ebase, EB)], didx)
            for k in range(EB // 16):
                sl = pl.ds(k * 16, 16)
                sidx[sl] = sidx[sl] + hoff
                dadj[sl] = didx[sl] + hoff
            cp1 = pltpu.async_copy(g_hbm.at[sidx], gbuf, sem1)
            cp2 = pltpu.async_copy(ad_hbm.at[dadj], abuf, sem2)
            cp1.wait()
            cp2.wait()
            for j in range(EB):
                t = gbuf[j, D:2 * D] + abuf[j, :]
                w = jnp.exp(jnp.maximum(t, 0.2 * t))
                sbuf[j, 0:D] = w * gbuf[j, 0:D]
                sbuf[j, D:2 * D] = w
            pltpu.sync_copy(sbuf, slab.at[didx], add=True)
            return c
        lax.fori_loop(0, NBLK, edge_blk, 0)
        plsc.subcore_barrier()

        for k in range(KCH):
            chunk = sid + 16 * k

            @pl.when(chunk < NCHUNK)
            def _():
                rr = chunk * RCHUNK
                pltpu.sync_copy(slab.at[pl.ds(rr, RCHUNK)], stage)
                pltpu.sync_copy(stage, u_hbm.at[pl.ds(hoff + rr, RCHUNK)])
                pltpu.sync_copy(zbuf, slab.at[pl.ds(rr, RCHUNK)])
        plsc.subcore_barrier()
        return carry

    lax.fori_loop(0, 4, round_body, 0)


def _edges(g_flat, ad_flat, src_list, dst_list):
    mesh = plsc.VectorSubcoreMesh(core_axis_name="c", subcore_axis_name="s")
    f = pl.kernel(
        _edge_kernel_body, mesh=mesh,
        compiler_params=pltpu.CompilerParams(use_tc_tiling_on_sc=False),
        out_type=jax.ShapeDtypeStruct((H * N, 2 * D), jnp.float32),
        scratch_types=[
            pltpu.VMEM((EB,), jnp.int32),
            pltpu.VMEM((EB,), jnp.int32),
            pltpu.VMEM((EB,), jnp.int32),
            pltpu.VMEM((EB, 2 * D), jnp.float32),
            pltpu.VMEM((EB, D), jnp.float32),
            pltpu.VMEM((EB, 2 * D), jnp.float32),
            pltpu.VMEM((RCHUNK, 2 * D), jnp.float32),
            pltpu.VMEM((RCHUNK, 2 * D), jnp.float32),
            pltpu.VMEM_SHARED((N, 2 * D), jnp.float32),
            pltpu.SemaphoreType.DMA,
            pltpu.SemaphoreType.DMA,
        ],
    )
    return f(g_flat, ad_flat, src_list, dst_list)


def _final_body(u_ref, hp_ref, ws_ref, bias_ref, wy1_ref, by1_ref,
                wy0_ref, by0_ref, wc1_ref, bc1_ref, wc2_ref, bc2_ref,
                ce_ref, pr_ref):
    parts = []
    for hh in range(H):
        wself = ws_ref[:, hh:hh + 1]
        num = u_ref[hh, :, 0:D] + wself * hp_ref[:, hh * D:(hh + 1) * D]
        den = u_ref[hh, :, D:2 * D] + wself
        parts.append(num / den)
    h2 = jnp.concatenate(parts, axis=1) + bias_ref[...]
    y1 = jnp.dot(h2, wy1_ref[...], preferred_element_type=jnp.float32) + by1_ref[...]
    y0 = jnp.dot(h2, wy0_ref[...], preferred_element_type=jnp.float32) + by0_ref[...]
    ce_ref[...] = y1 - y0
    z = jnp.maximum(jnp.dot(h2, wc1_ref[...],
                            preferred_element_type=jnp.float32) + bc1_ref[...], 0.0)
    p = jnp.dot(z, wc2_ref[...], preferred_element_type=jnp.float32) + bc2_ref[...]
    pr_ref[...] = jax.nn.sigmoid(p)


def _final(u3, hp, ws, bias, wy1t, by1, wy0t, by0, wc1t, bc1, wc2t, bc2):
    return pl.pallas_call(
        _final_body,
        grid=(GRID,),
        in_specs=[
            pl.BlockSpec((H, BN, 2 * D), lambda i: (0, i, 0)),
            pl.BlockSpec((BN, H * D), lambda i: (i, 0)),
            pl.BlockSpec((BN, H), lambda i: (i, 0)),
            pl.BlockSpec((1, H * D), lambda i: (0, 0)),
            pl.BlockSpec((H * D, 1), lambda i: (0, 0)),
            pl.BlockSpec((1, 1), lambda i: (0, 0)),
            pl.BlockSpec((H * D, 1), lambda i: (0, 0)),
            pl.BlockSpec((1, 1), lambda i: (0, 0)),
            pl.BlockSpec((H * D, D), lambda i: (0, 0)),
            pl.BlockSpec((1, D), lambda i: (0, 0)),
            pl.BlockSpec((D, 1), lambda i: (0, 0)),
            pl.BlockSpec((1, 1), lambda i: (0, 0)),
        ],
        out_specs=[
            pl.BlockSpec((BN, 1), lambda i: (i, 0)),
            pl.BlockSpec((BN, 1), lambda i: (i, 0)),
        ],
        out_shape=[
            jax.ShapeDtypeStruct((N, 1), jnp.float32),
            jax.ShapeDtypeStruct((N, 1), jnp.float32),
        ],
    )(u3, hp, ws, bias, wy1t, by1, wy0t, by0, wc1t, bc1, wc2t, bc2)


def kernel(x, edge_index, W_lin, b_lin, W_gat, att_src, att_dst, bias_gat,
           W_y1, b_y1, W_y0, b_y0, Wc1, bc1, Wc2, bc2):
    wlt = W_lin.T
    wgt = W_gat.T
    lane = jnp.arange(H * D, dtype=jnp.int32)
    ams = jnp.zeros((H * D, H), jnp.float32).at[lane, lane // D].set(
        att_src.reshape(H * D))
    amd = jnp.zeros((H * D, H), jnp.float32).at[lane, lane // D].set(
        att_dst.reshape(H * D))
    g3, ad3, hp, ws = _prep(x, wlt, b_lin.reshape(1, D), wgt, ams, amd)
    u_flat = _edges(g3.reshape(H * N, 2 * D), ad3.reshape(H * N, D),
                    edge_index[0], edge_index[1])
    ce, pr = _final(u_flat.reshape(H, N, 2 * D), hp, ws,
                    bias_gat.reshape(1, H * D),
                    W_y1.reshape(H * D, 1), b_y1.reshape(1, 1),
                    W_y0.reshape(H * D, 1), b_y0.reshape(1, 1),
                    Wc1.T, bc1.reshape(1, D), Wc2.reshape(D, 1),
                    bc2.reshape(1, 1))
    return (ce, pr)


# speedup vs baseline: 54.7366x; 2.1781x over previous
import jax
import jax.numpy as jnp
from jax import lax
from jax.experimental import pallas as pl
from jax.experimental.pallas import tpu as pltpu
from jax.experimental.pallas import tpu_sc as plsc

N = 50000
E = 800000
F = 128
D = 16
H = 8

BN = 2000
GRID = N // BN
EB = 80
NBLK_TOT = E // EB
NBLK = NBLK_TOT // 16
NBUF = 5
GROUPS = NBLK // NBUF
RCHUNK = 80
NCHUNK = N // RCHUNK
KCH = (NCHUNK + 15) // 16


def _leaky(x):
    return jnp.maximum(x, 0.2 * x)


def _prep_body(x_ref, wlt_ref, bl_ref, wgt_ref, ams_ref, amd_ref,
               g_ref, ad_ref, hp_ref, ws_ref):
    h = jnp.dot(x_ref[...], wlt_ref[...],
                preferred_element_type=jnp.float32) + bl_ref[...]
    hp = jnp.dot(h, wgt_ref[...], preferred_element_type=jnp.float32)
    asrc = jnp.dot(hp, ams_ref[...], preferred_element_type=jnp.float32)
    adst = jnp.dot(hp, amd_ref[...], preferred_element_type=jnp.float32)
    hp_ref[...] = hp
    ws_ref[...] = jnp.exp(_leaky(asrc + adst))
    ones = jnp.ones((1, D), dtype=jnp.float32)
    for hh in range(H):
        g_ref[hh, :, 0:D] = hp[:, hh * D:(hh + 1) * D]
        g_ref[hh, :, D:2 * D] = asrc[:, hh:hh + 1] * ones
        ad_ref[hh, :, :] = adst[:, hh:hh + 1] * ones


def _prep(x, wlt, bl, wgt, ams, amd):
    return pl.pallas_call(
        _prep_body,
        grid=(GRID,),
        in_specs=[
            pl.BlockSpec((BN, F), lambda i: (i, 0)),
            pl.BlockSpec((F, D), lambda i: (0, 0)),
            pl.BlockSpec((1, D), lambda i: (0, 0)),
            pl.BlockSpec((D, H * D), lambda i: (0, 0)),
            pl.BlockSpec((F, H), lambda i: (0, 0)),
            pl.BlockSpec((F, H), lambda i: (0, 0)),
        ],
        out_specs=[
            pl.BlockSpec((H, BN, 2 * D), lambda i: (0, i, 0)),
            pl.BlockSpec((H, BN, D), lambda i: (0, i, 0)),
            pl.BlockSpec((BN, H * D), lambda i: (i, 0)),
            pl.BlockSpec((BN, H), lambda i: (i, 0)),
        ],
        out_shape=[
            jax.ShapeDtypeStruct((H, N, 2 * D), jnp.float32),
            jax.ShapeDtypeStruct((H, N, D), jnp.float32),
            jax.ShapeDtypeStruct((N, H * D), jnp.float32),
            jax.ShapeDtypeStruct((N, H), jnp.float32),
        ],
    )(x, wlt, bl, wgt, ams, amd)


def _edge_kernel_body(g_hbm, ad_hbm, src_hbm, dst_hbm, u_hbm,
                      sidx, didx, dadj, gbuf, abuf, sbuf, stage, zbuf,
                      slab, semi, semg, sema):
    cid = lax.axis_index("c")
    sid = lax.axis_index("s")

    for j in range(RCHUNK):
        zbuf[j, 0:D] = jnp.zeros((D,), jnp.float32)
        zbuf[j, D:2 * D] = jnp.zeros((D,), jnp.float32)
    for k in range(KCH):
        chunk = sid + 16 * k

        @pl.when(chunk < NCHUNK)
        def _():
            pltpu.sync_copy(zbuf, slab.at[pl.ds(chunk * RCHUNK, RCHUNK)])
    plsc.subcore_barrier()

    def _issue_idx(b, blk):
        ebase = (blk * 16 + sid) * EB
        pltpu.async_copy(src_hbm.at[pl.ds(ebase, EB)], sidx.at[b], semi.at[b])
        pltpu.async_copy(dst_hbm.at[pl.ds(ebase, EB)], didx.at[b], semi.at[b])

    def _wait_idx(b, blk):
        ebase = (blk * 16 + sid) * EB
        pltpu.make_async_copy(
            src_hbm.at[pl.ds(ebase, EB)], sidx.at[b], semi.at[b]).wait()
        pltpu.make_async_copy(
            dst_hbm.at[pl.ds(ebase, EB)], didx.at[b], semi.at[b]).wait()

    def _adjust_and_gather(b, hoff):
        for k in range(EB // 16):
            sl = pl.ds(k * 16, 16)
            sidx[b, sl] = sidx[b, sl] + hoff
            dadj[b, sl] = didx[b, sl] + hoff
        pltpu.async_copy(g_hbm.at[sidx.at[b]], gbuf.at[b], semg.at[b])
        pltpu.async_copy(ad_hbm.at[dadj.at[b]], abuf.at[b], sema.at[b])

    def _process(b):
        pltpu.make_async_copy(
            g_hbm.at[sidx.at[b]], gbuf.at[b], semg.at[b]).wait()
        pltpu.make_async_copy(
            ad_hbm.at[dadj.at[b]], abuf.at[b], sema.at[b]).wait()
        for j in range(EB):
            t = gbuf[b, j, D:2 * D] + abuf[b, j, :]
            w = jnp.exp(jnp.maximum(t, 0.2 * t))
            sbuf[j, 0:D] = w * gbuf[b, j, 0:D]
            sbuf[j, D:2 * D] = w
        pltpu.sync_copy(sbuf, slab.at[didx.at[b]], add=True)

    def round_body(r, carry):
        head = cid * 4 + r
        hoff = head * N

        for b in range(NBUF):
            _issue_idx(b, b)
        _wait_idx(0, 0)
        _adjust_and_gather(0, hoff)

        def group(gi, c):
            for b in range(NBUF):
                blk = gi * NBUF + b
                nb = (b + 1) % NBUF
                nblk = blk + 1

                @pl.when(nblk < NBLK)
                def _():
                    _wait_idx(nb, nblk)
                    _adjust_and_gather(nb, hoff)
                _process(b)

                @pl.when(blk + NBUF < NBLK)
                def _():
                    _issue_idx(b, blk + NBUF)
            return c
        lax.fori_loop(0, GROUPS, group, 0)
        plsc.subcore_barrier()

        for k in range(KCH):
            chunk = sid + 16 * k

            @pl.when(chunk < NCHUNK)
            def _():
                rr = chunk * RCHUNK
                pltpu.sync_copy(slab.at[pl.ds(rr, RCHUNK)], stage)
                pltpu.sync_copy(stage, u_hbm.at[pl.ds(hoff + rr, RCHUNK)])
                pltpu.sync_copy(zbuf, slab.at[pl.ds(rr, RCHUNK)])
        plsc.subcore_barrier()
        return carry

    lax.fori_loop(0, 4, round_body, 0)


def _edges(g_flat, ad_flat, src_list, dst_list):
    mesh = plsc.VectorSubcoreMesh(core_axis_name="c", subcore_axis_name="s")
    f = pl.kernel(
        _edge_kernel_body, mesh=mesh,
        compiler_params=pltpu.CompilerParams(use_tc_tiling_on_sc=False),
        out_type=jax.ShapeDtypeStruct((H * N, 2 * D), jnp.float32),
        scratch_types=[
            pltpu.VMEM((NBUF, EB), jnp.int32),
            pltpu.VMEM((NBUF, EB), jnp.int32),
            pltpu.VMEM((NBUF, EB), jnp.int32),
            pltpu.VMEM((NBUF, EB, 2 * D), jnp.float32),
            pltpu.VMEM((NBUF, EB, D), jnp.float32),
            pltpu.VMEM((EB, 2 * D), jnp.float32),
            pltpu.VMEM((RCHUNK, 2 * D), jnp.float32),
            pltpu.VMEM((RCHUNK, 2 * D), jnp.float32),
            pltpu.VMEM_SHARED((N, 2 * D), jnp.float32),
            pltpu.SemaphoreType.DMA((NBUF,)),
            pltpu.SemaphoreType.DMA((NBUF,)),
            pltpu.SemaphoreType.DMA((NBUF,)),
        ],
    )
    return f(g_flat, ad_flat, src_list, dst_list)


def _final_body(u_ref, hp_ref, ws_ref, bias_ref, wy1_ref, by1_ref,
                wy0_ref, by0_ref, wc1_ref, bc1_ref, wc2_ref, bc2_ref,
                ce_ref, pr_ref):
    parts = []
    for hh in range(H):
        wself = ws_ref[:, hh:hh + 1]
        num = u_ref[hh, :, 0:D] + wself * hp_ref[:, hh * D:(hh + 1) * D]
        den = u_ref[hh, :, D:2 * D] + wself
        parts.append(num / den)
    h2 = jnp.concatenate(parts, axis=1) + bias_ref[...]
    y1 = jnp.dot(h2, wy1_ref[...], preferred_element_type=jnp.float32) + by1_ref[...]
    y0 = jnp.dot(h2, wy0_ref[...], preferred_element_type=jnp.float32) + by0_ref[...]
    ce_ref[...] = y1 - y0
    z = jnp.maximum(jnp.dot(h2, wc1_ref[...],
                            preferred_element_type=jnp.float32) + bc1_ref[...], 0.0)
    p = jnp.dot(z, wc2_ref[...], preferred_element_type=jnp.float32) + bc2_ref[...]
    pr_ref[...] = jax.nn.sigmoid(p)


def _final(u3, hp, ws, bias, wy1t, by1, wy0t, by0, wc1t, bc1, wc2t, bc2):
    return pl.pallas_call(
        _final_body,
        grid=(GRID,),
        in_specs=[
            pl.BlockSpec((H, BN, 2 * D), lambda i: (0, i, 0)),
            pl.BlockSpec((BN, H * D), lambda i: (i, 0)),
            pl.BlockSpec((BN, H), lambda i: (i, 0)),
            pl.BlockSpec((1, H * D), lambda i: (0, 0)),
            pl.BlockSpec((H * D, 1), lambda i: (0, 0)),
            pl.BlockSpec((1, 1), lambda i: (0, 0)),
            pl.BlockSpec((H * D, 1), lambda i: (0, 0)),
            pl.BlockSpec((1, 1), lambda i: (0, 0)),
            pl.BlockSpec((H * D, D), lambda i: (0, 0)),
            pl.BlockSpec((1, D), lambda i: (0, 0)),
            pl.BlockSpec((D, 1), lambda i: (0, 0)),
            pl.BlockSpec((1, 1), lambda i: (0, 0)),
        ],
        out_specs=[
            pl.BlockSpec((BN, 1), lambda i: (i, 0)),
            pl.BlockSpec((BN, 1), lambda i: (i, 0)),
        ],
        out_shape=[
            jax.ShapeDtypeStruct((N, 1), jnp.float32),
            jax.ShapeDtypeStruct((N, 1), jnp.float32),
        ],
    )(u3, hp, ws, bias, wy1t, by1, wy0t, by0, wc1t, bc1, wc2t, bc2)


def kernel(x, edge_index, W_lin, b_lin, W_gat, att_src, att_dst, bias_gat,
           W_y1, b_y1, W_y0, b_y0, Wc1, bc1, Wc2, bc2):
    wlt = W_lin.T
    wgt = W_gat.T
    lane = jnp.arange(H * D, dtype=jnp.int32)
    ams = jnp.zeros((H * D, H), jnp.float32).at[lane, lane // D].set(
        att_src.reshape(H * D))
    amd = jnp.zeros((H * D, H), jnp.float32).at[lane, lane // D].set(
        att_dst.reshape(H * D))
    g3, ad3, hp, ws = _prep(x, wlt, b_lin.reshape(1, D), wgt, ams, amd)
    u_flat = _edges(g3.reshape(H * N, 2 * D), ad3.reshape(H * N, D),
                    edge_index[0], edge_index[1])
    ce, pr = _final(u_flat.reshape(H, N, 2 * D), hp, ws,
                    bias_gat.reshape(1, H * D),
                    W_y1.reshape(H * D, 1), b_y1.reshape(1, 1),
                    W_y0.reshape(H * D, 1), b_y0.reshape(1, 1),
                    Wc1.T, bc1.reshape(1, D), Wc2.reshape(D, 1),
                    bc2.reshape(1, 1))
    return (ce, pr)


# speedup vs baseline: 59.2923x; 1.0832x over previous
import jax
import jax.numpy as jnp
from jax import lax
from jax.experimental import pallas as pl
from jax.experimental.pallas import tpu as pltpu
from jax.experimental.pallas import tpu_sc as plsc

N = 50000
E = 800000
F = 128
D = 16
H = 8

BN = 2000
GRID = N // BN
EB = 80
NBLK_TOT = E // EB
NBLK = NBLK_TOT // 16
NBUF = 5
GROUPS = NBLK // NBUF
RCHUNK = 80
NCHUNK = N // RCHUNK
KCH = (NCHUNK + 15) // 16


def _leaky(x):
    return jnp.maximum(x, 0.2 * x)


def _prep_body(x_ref, wlt_ref, bl_ref, wgt_ref, ams_ref, amd_ref,
               g_ref, ad_ref, hp_ref, ws_ref):
    h = jnp.dot(x_ref[...], wlt_ref[...],
                preferred_element_type=jnp.float32) + bl_ref[...]
    hp = jnp.dot(h, wgt_ref[...], preferred_element_type=jnp.float32)
    asrc = jnp.dot(hp, ams_ref[...], preferred_element_type=jnp.float32)
    adst = jnp.dot(hp, amd_ref[...], preferred_element_type=jnp.float32)
    hp_ref[...] = hp
    ws_ref[...] = jnp.exp(_leaky(asrc + adst))
    ones = jnp.ones((1, D), dtype=jnp.float32)
    for hh in range(H):
        g_ref[hh, :, 0:D] = hp[:, hh * D:(hh + 1) * D]
        g_ref[hh, :, D:2 * D] = asrc[:, hh:hh + 1] * ones
        ad_ref[hh, :, :] = adst[:, hh:hh + 1] * ones


def _prep(x, wlt, bl, wgt, ams, amd):
    return pl.pallas_call(
        _prep_body,
        grid=(GRID,),
        in_specs=[
            pl.BlockSpec((BN, F), lambda i: (i, 0)),
            pl.BlockSpec((F, D), lambda i: (0, 0)),
            pl.BlockSpec((1, D), lambda i: (0, 0)),
            pl.BlockSpec((D, H * D), lambda i: (0, 0)),
            pl.BlockSpec((F, H), lambda i: (0, 0)),
            pl.BlockSpec((F, H), lambda i: (0, 0)),
        ],
        out_specs=[
            pl.BlockSpec((H, BN, 2 * D), lambda i: (0, i, 0)),
            pl.BlockSpec((H, BN, D), lambda i: (0, i, 0)),
            pl.BlockSpec((BN, H * D), lambda i: (i, 0)),
            pl.BlockSpec((BN, H), lambda i: (i, 0)),
        ],
        out_shape=[
            jax.ShapeDtypeStruct((H, N, 2 * D), jnp.float32),
            jax.ShapeDtypeStruct((H, N, D), jnp.float32),
            jax.ShapeDtypeStruct((N, H * D), jnp.float32),
            jax.ShapeDtypeStruct((N, H), jnp.float32),
        ],
    )(x, wlt, bl, wgt, ams, amd)


def _edge_kernel_body(g_hbm, ad_hbm, src_hbm, dst_hbm, u_hbm,
                      sidx, didx, dadj, oidx, gbuf, abuf, stage, zbuf,
                      slab, semi, semg, sema, semsc):
    cid = lax.axis_index("c")
    sid = lax.axis_index("s")

    for j in range(RCHUNK):
        zbuf[j, 0:D] = jnp.zeros((D,), jnp.float32)
        zbuf[j, D:2 * D] = jnp.zeros((D,), jnp.float32)
    for k in range(KCH):
        chunk = sid + 16 * k

        @pl.when(chunk < NCHUNK)
        def _():
            pltpu.sync_copy(zbuf, slab.at[pl.ds(chunk * RCHUNK, RCHUNK)])
    plsc.subcore_barrier()

    def _issue_idx(b, blk):
        ebase = (blk * 16 + sid) * EB
        pltpu.async_copy(src_hbm.at[pl.ds(ebase, EB)], sidx.at[b], semi.at[b])
        pltpu.async_copy(dst_hbm.at[pl.ds(ebase, EB)], didx.at[b], semi.at[b])

    def _wait_idx(b, blk):
        ebase = (blk * 16 + sid) * EB
        pltpu.make_async_copy(
            src_hbm.at[pl.ds(ebase, EB)], sidx.at[b], semi.at[b]).wait()
        pltpu.make_async_copy(
            dst_hbm.at[pl.ds(ebase, EB)], didx.at[b], semi.at[b]).wait()

    def _adjust_and_gather(b, blk, hoff):
        @pl.when(blk >= NBUF)
        def _():
            pltpu.make_async_copy(
                gbuf.at[b], slab.at[oidx.at[b]], semsc.at[b]).wait()
        for k in range(EB // 16):
            sl = pl.ds(k * 16, 16)
            sidx[b, sl] = sidx[b, sl] + hoff
            dadj[b, sl] = didx[b, sl] + hoff
        pltpu.async_copy(g_hbm.at[sidx.at[b]], gbuf.at[b], semg.at[b])
        pltpu.async_copy(ad_hbm.at[dadj.at[b]], abuf.at[b], sema.at[b])

    def _process(b):
        pltpu.make_async_copy(
            g_hbm.at[sidx.at[b]], gbuf.at[b], semg.at[b]).wait()
        pltpu.make_async_copy(
            ad_hbm.at[dadj.at[b]], abuf.at[b], sema.at[b]).wait()
        for k in range(EB // 16):
            sl = pl.ds(k * 16, 16)
            oidx[b, sl] = didx[b, sl]
        for j in range(EB):
            t = gbuf[b, j, D:2 * D] + abuf[b, j, :]
            w = jnp.exp(jnp.maximum(t, 0.2 * t))
            gbuf[b, j, 0:D] = w * gbuf[b, j, 0:D]
            gbuf[b, j, D:2 * D] = w
        pltpu.async_copy(gbuf.at[b], slab.at[oidx.at[b]], semsc.at[b],
                         add=True)

    def round_body(r, carry):
        head = cid * 4 + r
        hoff = head * N

        for b in range(NBUF):
            _issue_idx(b, b)
        _wait_idx(0, 0)
        _adjust_and_gather(0, 0, hoff)

        def group(gi, c):
            for b in range(NBUF):
                blk = gi * NBUF + b
                nb = (b + 1) % NBUF
                nblk = blk + 1

                @pl.when(nblk < NBLK)
                def _():
                    _wait_idx(nb, nblk)
                    _adjust_and_gather(nb, nblk, hoff)
                _process(b)

                @pl.when(blk + NBUF < NBLK)
                def _():
                    _issue_idx(b, blk + NBUF)
            return c
        lax.fori_loop(0, GROUPS, group, 0)
        for b in range(NBUF):
            pltpu.make_async_copy(
                gbuf.at[b], slab.at[oidx.at[b]], semsc.at[b]).wait()
        plsc.subcore_barrier()

        for k in range(KCH):
            chunk = sid + 16 * k

            @pl.when(chunk < NCHUNK)
            def _():
                rr = chunk * RCHUNK
                pltpu.sync_copy(slab.at[pl.ds(rr, RCHUNK)], stage)
                pltpu.sync_copy(stage, u_hbm.at[pl.ds(hoff + rr, RCHUNK)])
                pltpu.sync_copy(zbuf, slab.at[pl.ds(rr, RCHUNK)])
        plsc.subcore_barrier()
        return carry

    lax.fori_loop(0, 4, round_body, 0)


def _edges(g_flat, ad_flat, src_list, dst_list):
    mesh = plsc.VectorSubcoreMesh(core_axis_name="c", subcore_axis_name="s")
    f = pl.kernel(
        _edge_kernel_body, mesh=mesh,
        compiler_params=pltpu.CompilerParams(use_tc_tiling_on_sc=False),
        out_type=jax.ShapeDtypeStruct((H * N, 2 * D), jnp.float32),
        scratch_types=[
            pltpu.VMEM((NBUF, EB), jnp.int32),
            pltpu.VMEM((NBUF, EB), jnp.int32),
            pltpu.VMEM((NBUF, EB), jnp.int32),
            pltpu.VMEM((NBUF, EB), jnp.int32),
            pltpu.VMEM((NBUF, EB, 2 * D), jnp.float32),
            pltpu.VMEM((NBUF, EB, D), jnp.float32),
            pltpu.VMEM((RCHUNK, 2 * D), jnp.float32),
            pltpu.VMEM((RCHUNK, 2 * D), jnp.float32),
            pltpu.VMEM_SHARED((N, 2 * D), jnp.float32),
            pltpu.SemaphoreType.DMA((NBUF,)),
            pltpu.SemaphoreType.DMA((NBUF,)),
            pltpu.SemaphoreType.DMA((NBUF,)),
            pltpu.SemaphoreType.DMA((NBUF,)),
        ],
    )
    return f(g_flat, ad_flat, src_list, dst_list)


def _final_body(u_ref, hp_ref, ws_ref, bias_ref, wy1_ref, by1_ref,
                wy0_ref, by0_ref, wc1_ref, bc1_ref, wc2_ref, bc2_ref,
                ce_ref, pr_ref):
    parts = []
    for hh in range(H):
        wself = ws_ref[:, hh:hh + 1]
        num = u_ref[hh, :, 0:D] + wself * hp_ref[:, hh * D:(hh + 1) * D]
        den = u_ref[hh, :, D:2 * D] + wself
        parts.append(num / den)
    h2 = jnp.concatenate(parts, axis=1) + bias_ref[...]
    y1 = jnp.dot(h2, wy1_ref[...], preferred_element_type=jnp.float32) + by1_ref[...]
    y0 = jnp.dot(h2, wy0_ref[...], preferred_element_type=jnp.float32) + by0_ref[...]
    ce_ref[...] = y1 - y0
    z = jnp.maximum(jnp.dot(h2, wc1_ref[...],
                            preferred_element_type=jnp.float32) + bc1_ref[...], 0.0)
    p = jnp.dot(z, wc2_ref[...], preferred_element_type=jnp.float32) + bc2_ref[...]
    pr_ref[...] = jax.nn.sigmoid(p)


def _final(u3, hp, ws, bias, wy1t, by1, wy0t, by0, wc1t, bc1, wc2t, bc2):
    return pl.pallas_call(
        _final_body,
        grid=(GRID,),
        in_specs=[
            pl.BlockSpec((H, BN, 2 * D), lambda i: (0, i, 0)),
            pl.BlockSpec((BN, H * D), lambda i: (i, 0)),
            pl.BlockSpec((BN, H), lambda i: (i, 0)),
            pl.BlockSpec((1, H * D), lambda i: (0, 0)),
            pl.BlockSpec((H * D, 1), lambda i: (0, 0)),
            pl.BlockSpec((1, 1), lambda i: (0, 0)),
            pl.BlockSpec((H * D, 1), lambda i: (0, 0)),
            pl.BlockSpec((1, 1), lambda i: (0, 0)),
            pl.BlockSpec((H * D, D), lambda i: (0, 0)),
            pl.BlockSpec((1, D), lambda i: (0, 0)),
            pl.BlockSpec((D, 1), lambda i: (0, 0)),
            pl.BlockSpec((1, 1), lambda i: (0, 0)),
        ],
        out_specs=[
            pl.BlockSpec((BN, 1), lambda i: (i, 0)),
            pl.BlockSpec((BN, 1), lambda i: (i, 0)),
        ],
        out_shape=[
            jax.ShapeDtypeStruct((N, 1), jnp.float32),
            jax.ShapeDtypeStruct((N, 1), jnp.float32),
        ],
    )(u3, hp, ws, bias, wy1t, by1, wy0t, by0, wc1t, bc1, wc2t, bc2)


def kernel(x, edge_index, W_lin, b_lin, W_gat, att_src, att_dst, bias_gat,
           W_y1, b_y1, W_y0, b_y0, Wc1, bc1, Wc2, bc2):
    wlt = W_lin.T
    wgt = W_gat.T
    lane = jnp.arange(H * D, dtype=jnp.int32)
    ams = jnp.zeros((H * D, H), jnp.float32).at[lane, lane // D].set(
        att_src.reshape(H * D))
    amd = jnp.zeros((H * D, H), jnp.float32).at[lane, lane // D].set(
        att_dst.reshape(H * D))
    g3, ad3, hp, ws = _prep(x, wlt, b_lin.reshape(1, D), wgt, ams, amd)
    u_flat = _edges(g3.reshape(H * N, 2 * D), ad3.reshape(H * N, D),
                    edge_index[0], edge_index[1])
    ce, pr = _final(u_flat.reshape(H, N, 2 * D), hp, ws,
                    bias_gat.reshape(1, H * D),
                    W_y1.reshape(H * D, 1), b_y1.reshape(1, 1),
                    W_y0.reshape(H * D, 1), b_y0.reshape(1, 1),
                    Wc1.T, bc1.reshape(1, D), Wc2.reshape(D, 1),
                    bc2.reshape(1, 1))
    return (ce, pr)
